# baseline (speedup 1.0000x reference)
import functools

import jax
import jax.numpy as jnp
from jax import lax
from jax.experimental import pallas as pl
from jax.experimental.pallas import tpu as pltpu
from jax.experimental.pallas import tpu_sc as plsc

F32 = jnp.float32
N_NODES = 50000
N_EDGES = 800000
NG = 64
NODE_IN = 128
FEA = 64
NL = 3

NTILES = 16
EPT = N_EDGES // NTILES
CH = 80
NCHUNK = EPT // CH
NPAD = 50048
RPT = NPAD // NTILES
WR = 136
NWR = RPT // WR

_lrelu = lambda v: jnp.where(v >= 0, v, 0.01 * v)


def _softplus16(b):
    u = jnp.exp(-jnp.abs(b))
    lg = u * (0.99999184 + u * (-0.49937278 + u * (0.32529598 + u * (
        -0.21029522 + u * (0.10150119 + u * -0.023979847)))))
    return jnp.maximum(b, 0.0) + lg


def _sigmoid16(a):
    return 1.0 / (1.0 + jnp.exp(-a))


def _sc_edge_layer(layer, Pe, Qe, Ee, idxe, Ps, Qs, Es, idxs):
    mesh = plsc.VectorSubcoreMesh(core_axis_name="c", subcore_axis_name="s")

    def body(pe_hbm, qe_hbm, ee_hbm, idxe_hbm, ps_hbm, qs_hbm, es_hbm, idxs_hbm,
             out_hbm,
             acc, raw_v, dstg_v, srcg_v, dsts_v,
             p_v, q_v, e_v, msg_v, zs_v,
             sem_ga, sem_gb, sem_ia, sem_ib, sem_s):
        c = lax.axis_index("c")
        s = lax.axis_index("s")
        zero = jnp.zeros((16,), F32)
        sem_g = (sem_ga, sem_gb)
        sem_i = (sem_ia, sem_ib)
        qoff = c * N_NODES
        eoff = (layer * 2 + c) * N_EDGES

        def clamp(k):
            return jnp.minimum(k, NCHUNK - 1)

        def run_graph(g, p_hbm, q_hbm, e_hbm, idx_hbm):
            def zbody(i, carry):
                zs_v[i, pl.ds(0, 16)] = zero
                zs_v[i, pl.ds(16, 16)] = zero
                return carry
            lax.fori_loop(0, WR, zbody, 0)
            for r in range(NWR):
                pltpu.sync_copy(zs_v, acc.at[pl.ds(s * RPT + r * WR, WR)])
            plsc.subcore_barrier()

            def issue_idx(p, k):
                row = s * NCHUNK + clamp(k)
                pltpu.async_copy(idx_hbm.at[row], raw_v.at[p], sem_i[p])

            def wait_idx(p):
                pltpu.make_async_copy(idx_hbm.at[0], raw_v.at[p], sem_i[p]).wait()

            def build_gidx(p):
                for b in range(CH // 16):
                    sl = pl.ds(b * 16, 16)
                    dstg_v[p, sl] = raw_v[p, 0, sl] + qoff
                    srcg_v[p, sl] = raw_v[p, 1, sl] + qoff

            def build_sidx(p):
                for b in range(CH // 16):
                    sl = pl.ds(b * 16, 16)
                    dsts_v[p, sl] = raw_v[p, 0, sl]

            def issue_gather(p, k):
                eb = s * EPT + clamp(k) * CH
                pltpu.async_copy(p_hbm.at[dstg_v.at[p]], p_v.at[p], sem_g[p])
                pltpu.async_copy(q_hbm.at[srcg_v.at[p]], q_v.at[p], sem_g[p])
                pltpu.async_copy(e_hbm.at[pl.ds(eoff + eb, CH)], e_v.at[p], sem_g[p])

            def wait_gather(p):
                pltpu.make_async_copy(p_hbm.at[pl.ds(0, CH)], p_v.at[p], sem_g[p]).wait()
                pltpu.make_async_copy(q_hbm.at[pl.ds(0, CH)], q_v.at[p], sem_g[p]).wait()
                pltpu.make_async_copy(e_hbm.at[pl.ds(0, CH)], e_v.at[p], sem_g[p]).wait()

            def compute(p):
                def ebody(j, cy):
                    pa0, pa1 = plsc.unpack(p_v[p, j, pl.ds(0, 32)],
                                           format=plsc.PackFormat.INTERLEAVED,
                                           preferred_element_type=F32)
                    ps0, ps1 = plsc.unpack(p_v[p, j, pl.ds(32, 32)],
                                           format=plsc.PackFormat.INTERLEAVED,
                                           preferred_element_type=F32)
                    qa0, qa1 = plsc.unpack(q_v[p, j, pl.ds(0, 32)],
                                           format=plsc.PackFormat.INTERLEAVED,
                                           preferred_element_type=F32)
                    qs0, qs1 = plsc.unpack(q_v[p, j, pl.ds(32, 32)],
                                           format=plsc.PackFormat.INTERLEAVED,
                                           preferred_element_type=F32)
                    a0 = pa0 + qa0 + e_v[p, j, pl.ds(0, 16)]
                    a1 = pa1 + qa1 + e_v[p, j, pl.ds(16, 16)]
                    b0 = ps0 + qs0 + e_v[p, j, pl.ds(32, 16)]
                    b1 = ps1 + qs1 + e_v[p, j, pl.ds(48, 16)]
                    msg_v[p, j, pl.ds(0, 16)] = _sigmoid16(a0) * _softplus16(b0)
                    msg_v[p, j, pl.ds(16, 16)] = _sigmoid16(a1) * _softplus16(b1)
                    return cy
                lax.fori_loop(0, CH, ebody, 0, unroll=4)

            def scatter(p):
                pltpu.async_copy(msg_v.at[p], acc.at[dsts_v.at[p]], sem_s, add=True)

            def wait_scatter(p):
                pltpu.make_async_copy(msg_v.at[p], acc.at[dsts_v.at[p]], sem_s).wait()

            def stage(p, k, first=False):
                q = 1 - p
                wait_idx(q)
                build_gidx(q)
                issue_gather(q, k + 1)
                issue_idx(p, k + 2)
                wait_gather(p)
                compute(p)
                if not first:
                    wait_scatter(q)
                build_sidx(q)
                scatter(p)

            issue_idx(0, 0)
            wait_idx(0)
            build_gidx(0)
            build_sidx(0)
            issue_gather(0, 0)
            issue_idx(1, 1)

            stage(0, 0, first=True)
            stage(1, 1)

            def pair2(i, carry):
                k = 2 + i * 2
                stage(0, k)
                stage(1, k + 1)
                return carry
            lax.fori_loop(0, (NCHUNK - 3) // 2, pair2, 0)
            stage(0, NCHUNK - 1)
            wait_gather(1)
            wait_idx(0)
            wait_scatter(0)

            plsc.subcore_barrier()
            base = g * 2 * NPAD + c * NPAD
            for r in range(NWR):
                pltpu.sync_copy(acc.at[pl.ds(s * RPT + r * WR, WR)], zs_v)
                pltpu.sync_copy(zs_v, out_hbm.at[pl.ds(base + s * RPT + r * WR, WR)])
            plsc.subcore_barrier()

        run_graph(0, pe_hbm, qe_hbm, ee_hbm, idxe_hbm)
        run_graph(1, ps_hbm, qs_hbm, es_hbm, idxs_hbm)

    f = pl.kernel(
        body,
        out_type=jax.ShapeDtypeStruct((4 * NPAD, 32), F32),
        mesh=mesh,
        scratch_types=[
            pltpu.VMEM_SHARED((NPAD, 32), F32),
            pltpu.VMEM((2, 2, CH), jnp.int32),
            pltpu.VMEM((2, CH), jnp.int32),
            pltpu.VMEM((2, CH), jnp.int32),
            pltpu.VMEM((2, CH), jnp.int32),
            pltpu.VMEM((2, CH, FEA), jnp.bfloat16),
            pltpu.VMEM((2, CH, FEA), jnp.bfloat16),
            pltpu.VMEM((2, CH, FEA), F32),
            pltpu.VMEM((2, CH, 32), F32),
            pltpu.VMEM((WR, 32), F32),
            pltpu.SemaphoreType.DMA,
            pltpu.SemaphoreType.DMA,
            pltpu.SemaphoreType.DMA,
            pltpu.SemaphoreType.DMA,
            pltpu.SemaphoreType.DMA,
        ],
        compiler_params=pltpu.CompilerParams(use_tc_tiling_on_sc=False, needs_layout_passes=False),
    )
    return f(Pe, Qe, Ee, idxe, Ps, Qs, Es, idxs)


def _sc_count(idxe, idxs):
    mesh = plsc.VectorSubcoreMesh(core_axis_name="c", subcore_axis_name="s")
    NJ = NCHUNK // 2

    def body(idxe_hbm, idxs_hbm, out_hbm, acc, raw_v, sidx_v, ones_v, zs_v,
             sem_i0, sem_i1, sem_s):
        c = lax.axis_index("c")
        s = lax.axis_index("s")
        zero = jnp.zeros((16,), F32)
        one = jnp.ones((16,), F32)
        sem_i = (sem_i0, sem_i1)

        def obody(i, carry):
            ones_v[i, pl.ds(0, 16)] = one
            return carry
        lax.fori_loop(0, CH, obody, 0)

        def run_graph(g, idx_hbm):
            def zbody(i, carry):
                zs_v[i, pl.ds(0, 16)] = zero
                return carry
            lax.fori_loop(0, WR, zbody, 0)
            for r in range(NWR):
                pltpu.sync_copy(zs_v, acc.at[pl.ds(s * RPT + r * WR, WR)])
            plsc.subcore_barrier()

            def row_of(j):
                return s * NCHUNK + jnp.minimum(c + 2 * j, NCHUNK - 1)

            def issue_idx(p, j):
                pltpu.async_copy(idx_hbm.at[row_of(j), 0], raw_v.at[p], sem_i[p])

            def wait_idx(p):
                pltpu.make_async_copy(idx_hbm.at[0, 0], raw_v.at[p], sem_i[p]).wait()

            def scatter(p):
                pltpu.async_copy(ones_v, acc.at[sidx_v.at[p]], sem_s, add=True)

            def wait_scatter(p):
                pltpu.make_async_copy(ones_v, acc.at[sidx_v.at[p]], sem_s).wait()

            def stage(p, j, first=False):
                wait_idx(p)
                if not first:
                    wait_scatter(p)
                for b in range(CH // 16):
                    sl = pl.ds(b * 16, 16)
                    sidx_v[p, sl] = raw_v[p, sl]
                scatter(p)
                issue_idx(p, j + 2)

            issue_idx(0, 0)
            issue_idx(1, 1)
            stage(0, 0, first=True)
            stage(1, 1, first=True)

            def pair(i, carry):
                j = 2 + i * 2
                stage(0, j)
                stage(1, j + 1)
                return carry
            lax.fori_loop(0, (NJ - 2) // 2, pair, 0)

            @pl.when(c == 0)
            def _():
                stage(0, NJ)
            wait_idx(0)
            wait_idx(1)
            wait_scatter(0)
            wait_scatter(1)

            plsc.subcore_barrier()
            base = g * 2 * NPAD + c * NPAD
            for r in range(NWR):
                pltpu.sync_copy(acc.at[pl.ds(s * RPT + r * WR, WR)], zs_v)
                pltpu.sync_copy(zs_v, out_hbm.at[pl.ds(base + s * RPT + r * WR, WR)])
            plsc.subcore_barrier()

        run_graph(0, idxe_hbm)
        run_graph(1, idxs_hbm)

    f = pl.kernel(
        body,
        out_type=jax.ShapeDtypeStruct((4 * NPAD, 16), F32),
        mesh=mesh,
        scratch_types=[
            pltpu.VMEM_SHARED((NPAD, 16), F32),
            pltpu.VMEM((2, CH), jnp.int32),
            pltpu.VMEM((2, CH), jnp.int32),
            pltpu.VMEM((CH, 16), F32),
            pltpu.VMEM((WR, 16), F32),
            pltpu.SemaphoreType.DMA,
            pltpu.SemaphoreType.DMA,
            pltpu.SemaphoreType.DMA,
        ],
        compiler_params=pltpu.CompilerParams(use_tc_tiling_on_sc=False, needs_layout_passes=False),
    )
    return f(idxe, idxs)


BLK = 2000
NBLK = N_NODES // BLK
NEBLK = N_EDGES // BLK


def _embed_pq(x, W, b, Wcat0):
    def body(x_ref, w_ref, b_ref, wc_ref, o_ref, p_ref, q_ref):
        x0 = _lrelu(jnp.dot(x_ref[...], w_ref[...],
                            preferred_element_type=F32) + b_ref[...])
        o_ref[...] = x0
        z = jnp.dot(x0, wc_ref[...], preferred_element_type=F32)
        p_ref[0] = z[:, 0:64].astype(jnp.bfloat16)
        p_ref[1] = z[:, 64:128].astype(jnp.bfloat16)
        q_ref[0] = z[:, 128:192].astype(jnp.bfloat16)
        q_ref[1] = z[:, 192:256].astype(jnp.bfloat16)
    return pl.pallas_call(
        body,
        grid=(NBLK,),
        in_specs=[pl.BlockSpec((BLK, NODE_IN), lambda j: (j, 0)),
                  pl.BlockSpec((NODE_IN, FEA), lambda j: (0, 0)),
                  pl.BlockSpec((1, FEA), lambda j: (0, 0)),
                  pl.BlockSpec((FEA, 4 * FEA), lambda j: (0, 0))],
        out_specs=[pl.BlockSpec((BLK, FEA), lambda j: (j, 0)),
                   pl.BlockSpec((2, BLK, FEA), lambda j: (0, j, 0)),
                   pl.BlockSpec((2, BLK, FEA), lambda j: (0, j, 0))],
        out_shape=[jax.ShapeDtypeStruct((N_NODES, FEA), F32),
                   jax.ShapeDtypeStruct((2, N_NODES, FEA), jnp.bfloat16),
                   jax.ShapeDtypeStruct((2, N_NODES, FEA), jnp.bfloat16)],
    )(x, W, b.reshape(1, FEA), Wcat0)


def _edge_E(attr, eW, eb, Wcomb, bcomb):
    def body(a_ref, ew_ref, ebias_ref, wc_ref, bc_ref, o_ref):
        e = _lrelu(jnp.dot(a_ref[...], ew_ref[...],
                           preferred_element_type=F32) + ebias_ref[...])
        for i in range(NL):
            z = jnp.dot(e, wc_ref[i], preferred_element_type=F32) + bc_ref[i]
            ef, es = z[:, :64], z[:, 64:]
            o_ref[i, 0] = jnp.concatenate([ef[:, :32], es[:, :32]], axis=1)
            o_ref[i, 1] = jnp.concatenate([ef[:, 32:], es[:, 32:]], axis=1)
    return pl.pallas_call(
        body,
        grid=(NEBLK,),
        in_specs=[pl.BlockSpec((BLK, 16), lambda j: (j, 0)),
                  pl.BlockSpec((16, FEA), lambda j: (0, 0)),
                  pl.BlockSpec((1, FEA), lambda j: (0, 0)),
                  pl.BlockSpec((NL, FEA, 2 * FEA), lambda j: (0, 0, 0)),
                  pl.BlockSpec((NL, 1, 2 * FEA), lambda j: (0, 0, 0))],
        out_specs=pl.BlockSpec((NL, 2, BLK, FEA), lambda j: (0, 0, j, 0)),
        out_shape=jax.ShapeDtypeStruct((NL, 2, N_EDGES, FEA), F32),
    )(attr, eW, eb.reshape(1, FEA), Wcomb, bcomb)


def _stats(agg0, agg1, cnt0, cnt1):
    def body(a0_ref, a1_ref, c0_ref, c1_ref, o_ref):
        j = pl.program_id(0)
        inv = 1.0 / jnp.maximum(c0_ref[:, 0:1] + c1_ref[:, 0:1], 1.0)
        a = jnp.concatenate([a0_ref[...], a1_ref[...]], axis=1) * inv

        @pl.when(j == 0)
        def _():
            o_ref[...] = jnp.zeros_like(o_ref)
        o_ref[0:1, :] = o_ref[0:1, :] + jnp.sum(a, axis=0, keepdims=True)
        o_ref[1:2, :] = o_ref[1:2, :] + jnp.sum(a * a, axis=0, keepdims=True)
    return pl.pallas_call(
        body,
        grid=(NBLK,),
        in_specs=[pl.BlockSpec((BLK, 32), lambda j: (j, 0)),
                  pl.BlockSpec((BLK, 32), lambda j: (j, 0)),
                  pl.BlockSpec((BLK, 16), lambda j: (j, 0)),
                  pl.BlockSpec((BLK, 16), lambda j: (j, 0))],
        out_specs=pl.BlockSpec((8, FEA), lambda j: (0, 0)),
        out_shape=jax.ShapeDtypeStruct((8, FEA), F32),
    )(agg0, agg1, cnt0, cnt1)


def _apply_pq(agg0, agg1, cnt0, cnt1, x, st, gamma, beta, Wcat_next):
    with_pq = Wcat_next is not None

    def body(a0_ref, a1_ref, c0_ref, c1_ref, x_ref, s_ref, g_ref, b_ref, *rest):
        if with_pq:
            w_ref, o_ref, p_ref, q_ref = rest
        else:
            (o_ref,) = rest
        inv = 1.0 / jnp.maximum(c0_ref[:, 0:1] + c1_ref[:, 0:1], 1.0)
        a = jnp.concatenate([a0_ref[...], a1_ref[...]], axis=1) * inv
        mean = s_ref[0:1, :] / N_NODES
        var = s_ref[1:2, :] / N_NODES - mean * mean
        xn = (a - mean) / jnp.sqrt(var + 1e-5) * g_ref[...] + b_ref[...] + x_ref[...]
        o_ref[...] = xn
        if with_pq:
            z = jnp.dot(xn, w_ref[...], preferred_element_type=F32)
            p_ref[0] = z[:, 0:64].astype(jnp.bfloat16)
            p_ref[1] = z[:, 64:128].astype(jnp.bfloat16)
            q_ref[0] = z[:, 128:192].astype(jnp.bfloat16)
            q_ref[1] = z[:, 192:256].astype(jnp.bfloat16)

    in_specs = [pl.BlockSpec((BLK, 32), lambda j: (j, 0)),
                pl.BlockSpec((BLK, 32), lambda j: (j, 0)),
                pl.BlockSpec((BLK, 16), lambda j: (j, 0)),
                pl.BlockSpec((BLK, 16), lambda j: (j, 0)),
                pl.BlockSpec((BLK, FEA), lambda j: (j, 0)),
                pl.BlockSpec((8, FEA), lambda j: (0, 0)),
                pl.BlockSpec((1, FEA), lambda j: (0, 0)),
                pl.BlockSpec((1, FEA), lambda j: (0, 0))]
    out_specs = [pl.BlockSpec((BLK, FEA), lambda j: (j, 0))]
    out_shape = [jax.ShapeDtypeStruct((N_NODES, FEA), F32)]
    args = [agg0, agg1, cnt0, cnt1, x, st, gamma.reshape(1, FEA), beta.reshape(1, FEA)]
    if with_pq:
        in_specs.append(pl.BlockSpec((FEA, 4 * FEA), lambda j: (0, 0)))
        out_specs += [pl.BlockSpec((2, BLK, FEA), lambda j: (0, j, 0)),
                      pl.BlockSpec((2, BLK, FEA), lambda j: (0, j, 0))]
        out_shape += [jax.ShapeDtypeStruct((2, N_NODES, FEA), jnp.bfloat16),
                      jax.ShapeDtypeStruct((2, N_NODES, FEA), jnp.bfloat16)]
        args.append(Wcat_next)
    res = pl.pallas_call(
        body,
        grid=(NBLK,),
        in_specs=in_specs,
        out_specs=out_specs,
        out_shape=out_shape,
    )(*args)
    return res if with_pq else (res[0], None, None)


def _gate_max(x, batch, gW1, gb1, gW2, gb2):
    def body(x_ref, b_ref, w1_ref, b1_ref, w2_ref, b2_ref, gate_ref, m_ref):
        j = pl.program_id(0)
        g1 = jnp.dot(x_ref[...], w1_ref[...], preferred_element_type=F32) + b1_ref[...]
        gate = jnp.dot(g1, w2_ref[...], preferred_element_type=F32) + b2_ref[...]
        gate_ref[...] = gate
        gid = lax.broadcasted_iota(jnp.int32, (1, NG), 1)
        M = b_ref[...] == gid
        masked = jnp.where(M, gate, -3e38)
        bm = jnp.max(masked, axis=0, keepdims=True)

        @pl.when(j == 0)
        def _():
            m_ref[...] = jnp.full_like(m_ref, -3e38)
        m_ref[...] = jnp.maximum(m_ref[...], bm)
    return pl.pallas_call(
        body,
        grid=(NBLK,),
        in_specs=[pl.BlockSpec((BLK, FEA), lambda j: (j, 0)),
                  pl.BlockSpec((BLK, 1), lambda j: (j, 0)),
                  pl.BlockSpec((FEA, FEA), lambda j: (0, 0)),
                  pl.BlockSpec((1, FEA), lambda j: (0, 0)),
                  pl.BlockSpec((FEA, 1), lambda j: (0, 0)),
                  pl.BlockSpec((1, 1), lambda j: (0, 0))],
        out_specs=[pl.BlockSpec((BLK, 1), lambda j: (j, 0)),
                   pl.BlockSpec((1, NG), lambda j: (0, 0))],
        out_shape=[jax.ShapeDtypeStruct((N_NODES, 1), F32),
                   jax.ShapeDtypeStruct((1, NG), F32)],
    )(x, batch, gW1, gb1.reshape(1, FEA), gW2, gb2.reshape(1, 1))


def _pool_acc(x, batch, gate, m, nW1, nb1, nW2, nb2):
    def body(x_ref, b_ref, gate_ref, m_ref, w1_ref, b1_ref, w2_ref, b2_ref,
             s_ref, n_ref):
        j = pl.program_id(0)
        h1 = jnp.dot(x_ref[...], w1_ref[...], preferred_element_type=F32) + b1_ref[...]
        h = jnp.dot(h1, w2_ref[...], preferred_element_type=F32) + b2_ref[...]
        gid = lax.broadcasted_iota(jnp.int32, (1, NG), 1)
        M = (b_ref[...] == gid).astype(F32)
        mb = jnp.sum(M * m_ref[...], axis=1, keepdims=True)
        g = jnp.exp(gate_ref[...] - mb)
        Mg = M * g
        s_blk = jnp.sum(Mg, axis=0, keepdims=True)
        n_blk = lax.dot_general(h, Mg, (((0,), (0,)), ((), ())),
                                preferred_element_type=F32)

        @pl.when(j == 0)
        def _():
            s_ref[...] = jnp.zeros_like(s_ref)
            n_ref[...] = jnp.zeros_like(n_ref)
        s_ref[...] = s_ref[...] + s_blk
        n_ref[...] = n_ref[...] + n_blk
    return pl.pallas_call(
        body,
        grid=(NBLK,),
        in_specs=[pl.BlockSpec((BLK, FEA), lambda j: (j, 0)),
                  pl.BlockSpec((BLK, 1), lambda j: (j, 0)),
                  pl.BlockSpec((BLK, 1), lambda j: (j, 0)),
                  pl.BlockSpec((1, NG), lambda j: (0, 0)),
                  pl.BlockSpec((FEA, FEA), lambda j: (0, 0)),
                  pl.BlockSpec((1, FEA), lambda j: (0, 0)),
                  pl.BlockSpec((FEA, FEA), lambda j: (0, 0)),
                  pl.BlockSpec((1, FEA), lambda j: (0, 0))],
        out_specs=[pl.BlockSpec((1, NG), lambda j: (0, 0)),
                   pl.BlockSpec((FEA, NG), lambda j: (0, 0))],
        out_shape=[jax.ShapeDtypeStruct((1, NG), F32),
                   jax.ShapeDtypeStruct((FEA, NG), F32)],
    )(x, batch, gate, m, nW1, nb1.reshape(1, FEA), nW2, nb2.reshape(1, FEA))


def _heads(Ne, se, Ns, ss, h_W, h_b, out_W, out_b, hexp_W, hexp_b):
    def body(ne_ref, se_ref, ns_ref, ss_ref, hw_ref, hb_ref, ow_ref, ob_ref,
             xw_ref, xb_ref, eo_ref, so_ref):
        pe = ne_ref[...] / jnp.maximum(se_ref[...], 1e-30)
        ps = ns_ref[...] / jnp.maximum(ss_ref[...], 1e-30)
        so1 = _lrelu(lax.dot_general(hw_ref[...], ps, (((0,), (0,)), ((), ())),
                                     preferred_element_type=F32) + hb_ref[...])
        so_ref[...] = lax.dot_general(ow_ref[...], so1, (((0,), (0,)), ((), ())),
                                      preferred_element_type=F32) + ob_ref[...]
        eo_ref[...] = lax.dot_general(xw_ref[...], pe, (((0,), (0,)), ((), ())),
                                      preferred_element_type=F32) + xb_ref[...]
    return pl.pallas_call(
        body,
        out_shape=[jax.ShapeDtypeStruct((1, NG), F32),
                   jax.ShapeDtypeStruct((1, NG), F32)],
    )(Ne, se, Ns, ss, h_W, h_b.reshape(FEA, 1), out_W, out_b.reshape(1, 1),
      hexp_W, hexp_b.reshape(1, 1))


def kernel(exp_x, exp_edge_index, exp_edge_attr, exp_batch, sim_x,
           sim_edge_index, sim_edge_attr, sim_batch, node_W, node_b, edge_W,
           edge_b, linf_W, linf_b, lins_W, lins_b, bn_g, bn_b, gate_W1,
           gate_b1, gate_W2, gate_b2, nn_W1, nn_b1, nn_W2, nn_b2, h_W, h_b,
           out_W, out_b, hexp_W, hexp_b):
    dst_e = exp_edge_index[1]
    src_e = exp_edge_index[0]
    dst_s = sim_edge_index[1]
    src_s = sim_edge_index[0]
    batch_e = exp_batch.reshape(N_NODES, 1)
    batch_s = sim_batch.reshape(N_NODES, 1)

    def _ilv(base):
        out = []
        for j in range(16):
            out += [base + j, base + 16 + j]
        return out
    perm_pq = []
    for sec in (0, 128):
        for c2 in (0, 1):
            perm_pq += _ilv(sec + c2 * 32) + _ilv(sec + 64 + c2 * 32)
    perm_pq = jnp.array(perm_pq, dtype=jnp.int32)
    Wcat = [jnp.concatenate([linf_W[i, 0:64], lins_W[i, 0:64],
                             linf_W[i, 64:128], lins_W[i, 64:128]],
                            axis=1)[:, perm_pq]
            for i in range(NL)]
    Wcomb = jnp.stack([jnp.concatenate([linf_W[i, 128:192], lins_W[i, 128:192]],
                                       axis=1) for i in range(NL)])
    bcomb = jnp.stack([jnp.concatenate([linf_b[i], lins_b[i]]).reshape(1, 2 * FEA)
                       for i in range(NL)])

    idx_e = jnp.stack([dst_e.reshape(-1, CH), src_e.reshape(-1, CH)], axis=1)
    idx_s = jnp.stack([dst_s.reshape(-1, CH), src_s.reshape(-1, CH)], axis=1)
    xe, Pe, Qe = _embed_pq(exp_x, node_W, node_b, Wcat[0])
    xs, Ps, Qs = _embed_pq(sim_x, node_W, node_b, Wcat[0])
    Ee = _edge_E(exp_edge_attr, edge_W, edge_b, Wcomb, bcomb)
    Es = _edge_E(sim_edge_attr, edge_W, edge_b, Wcomb, bcomb)
    Ee = Ee.reshape(NL * 2 * N_EDGES, FEA)
    Es = Es.reshape(NL * 2 * N_EDGES, FEA)
    cnt = _sc_count(idx_e, idx_s)
    cnt_halves = dict(
        e=(cnt[0:NPAD], cnt[NPAD:2 * NPAD]),
        s=(cnt[2 * NPAD:3 * NPAD], cnt[3 * NPAD:]),
    )

    for i in range(NL):
        agg = _sc_edge_layer(i, Pe.reshape(2 * N_NODES, FEA),
                             Qe.reshape(2 * N_NODES, FEA), Ee, idx_e,
                             Ps.reshape(2 * N_NODES, FEA),
                             Qs.reshape(2 * N_NODES, FEA), Es, idx_s)
        Wn = Wcat[i + 1] if i + 1 < NL else None
        outs = dict()
        for g, tag, x in ((0, "e", xe), (1, "s", xs)):
            agg0 = agg[g * 2 * NPAD:g * 2 * NPAD + NPAD]
            agg1 = agg[g * 2 * NPAD + NPAD:(g + 1) * 2 * NPAD]
            c0, c1 = cnt_halves[tag]
            st = _stats(agg0, agg1, c0, c1)
            outs[tag] = _apply_pq(agg0, agg1, c0, c1, x, st,
                                  bn_g[i], bn_b[i], Wn)
        xe, Pe, Qe = outs["e"]
        xs, Ps, Qs = outs["s"]

    gate_e, m_e = _gate_max(xe, batch_e, gate_W1, gate_b1, gate_W2, gate_b2)
    gate_s, m_s = _gate_max(xs, batch_s, gate_W1, gate_b1, gate_W2, gate_b2)
    s_e, N_e = _pool_acc(xe, batch_e, gate_e, m_e, nn_W1, nn_b1, nn_W2, nn_b2)
    s_s, N_s = _pool_acc(xs, batch_s, gate_s, m_s, nn_W1, nn_b1, nn_W2, nn_b2)
    eo, so = _heads(N_e, s_e, N_s, s_s, h_W, h_b, out_W, out_b, hexp_W, hexp_b)
    return (eo.reshape(NG), so.reshape(NG))


# speedup vs baseline: 1.1261x; 1.1261x over previous
import functools

import jax
import jax.numpy as jnp
from jax import lax
from jax.experimental import pallas as pl
from jax.experimental.pallas import tpu as pltpu
from jax.experimental.pallas import tpu_sc as plsc

F32 = jnp.float32
N_NODES = 50000
N_EDGES = 800000
NG = 64
NODE_IN = 128
FEA = 64
NL = 3

NTILES = 16
EPT = N_EDGES // NTILES
CH = 80
NCHUNK = EPT // CH
NPAD = 50048
RPT = NPAD // NTILES
WR = 136
NWR = RPT // WR

_lrelu = lambda v: jnp.where(v >= 0, v, 0.01 * v)


def _softplus16(b):
    u = jnp.exp(-jnp.abs(b))
    lg = u * (0.99999184 + u * (-0.49937278 + u * (0.32529598 + u * (
        -0.21029522 + u * (0.10150119 + u * -0.023979847)))))
    return jnp.maximum(b, 0.0) + lg


def _sigmoid16(a):
    return 1.0 / (1.0 + jnp.exp(-a))


def _sc_edge_layer(layer, P, Q, E, idx):
    mesh = plsc.VectorSubcoreMesh(core_axis_name="c", subcore_axis_name="s")

    def body(p_hbm, q_hbm, e_hbm, idx_hbm, out_hbm,
             acc, raw_v, dstg_v, srcg_v, dsts_v,
             p_v, q_v, e_v, msg_v, zs_v,
             sem_ga, sem_gb, sem_ia, sem_ib, sem_s):
        c = lax.axis_index("c")
        s = lax.axis_index("s")
        zero = jnp.zeros((16,), F32)
        sem_g = (sem_ga, sem_gb)
        sem_i = (sem_ia, sem_ib)

        def zbody(i, carry):
            zs_v[i, pl.ds(0, 16)] = zero
            zs_v[i, pl.ds(16, 16)] = zero
            return carry
        lax.fori_loop(0, WR, zbody, 0)
        for r in range(NWR):
            pltpu.sync_copy(zs_v, acc.at[pl.ds(s * RPT + r * WR, WR)])
        plsc.subcore_barrier()

        qoff = c * N_NODES
        eoff = (layer * 2 + c) * N_EDGES

        def clamp(k):
            return jnp.minimum(k, NCHUNK - 1)

        def issue_idx(p, k):
            row = s * NCHUNK + clamp(k)
            pltpu.async_copy(idx_hbm.at[row], raw_v.at[p], sem_i[p])

        def wait_idx(p):
            pltpu.make_async_copy(idx_hbm.at[0], raw_v.at[p], sem_i[p]).wait()

        def build_gidx(p):
            for b in range(CH // 16):
                sl = pl.ds(b * 16, 16)
                dstg_v[p, sl] = raw_v[p, 0, sl] + qoff
                srcg_v[p, sl] = raw_v[p, 1, sl] + qoff

        def build_sidx(p):
            for b in range(CH // 16):
                sl = pl.ds(b * 16, 16)
                dsts_v[p, sl] = raw_v[p, 0, sl]

        def issue_gather(p, k):
            eb = s * EPT + clamp(k) * CH
            pltpu.async_copy(p_hbm.at[dstg_v.at[p]], p_v.at[p], sem_g[p])
            pltpu.async_copy(q_hbm.at[srcg_v.at[p]], q_v.at[p], sem_g[p])
            pltpu.async_copy(e_hbm.at[pl.ds(eoff + eb, CH)], e_v.at[p], sem_g[p])

        def wait_gather(p):
            pltpu.make_async_copy(p_hbm.at[pl.ds(0, CH)], p_v.at[p], sem_g[p]).wait()
            pltpu.make_async_copy(q_hbm.at[pl.ds(0, CH)], q_v.at[p], sem_g[p]).wait()
            pltpu.make_async_copy(e_hbm.at[pl.ds(0, CH)], e_v.at[p], sem_g[p]).wait()

        def compute(p):
            def ebody(j, cy):
                pa0, pa1 = plsc.unpack(p_v[p, j, pl.ds(0, 32)],
                                       format=plsc.PackFormat.INTERLEAVED,
                                       preferred_element_type=F32)
                ps0, ps1 = plsc.unpack(p_v[p, j, pl.ds(32, 32)],
                                       format=plsc.PackFormat.INTERLEAVED,
                                       preferred_element_type=F32)
                qa0, qa1 = plsc.unpack(q_v[p, j, pl.ds(0, 32)],
                                       format=plsc.PackFormat.INTERLEAVED,
                                       preferred_element_type=F32)
                qs0, qs1 = plsc.unpack(q_v[p, j, pl.ds(32, 32)],
                                       format=plsc.PackFormat.INTERLEAVED,
                                       preferred_element_type=F32)
                a0 = pa0 + qa0 + e_v[p, j, pl.ds(0, 16)]
                a1 = pa1 + qa1 + e_v[p, j, pl.ds(16, 16)]
                b0 = ps0 + qs0 + e_v[p, j, pl.ds(32, 16)]
                b1 = ps1 + qs1 + e_v[p, j, pl.ds(48, 16)]
                msg_v[p, j, pl.ds(0, 16)] = _sigmoid16(a0) * _softplus16(b0)
                msg_v[p, j, pl.ds(16, 16)] = _sigmoid16(a1) * _softplus16(b1)
                return cy
            lax.fori_loop(0, CH, ebody, 0, unroll=4)

        def scatter(p):
            pltpu.async_copy(msg_v.at[p], acc.at[dsts_v.at[p]], sem_s, add=True)

        def wait_scatter(p):
            pltpu.make_async_copy(msg_v.at[p], acc.at[dsts_v.at[p]], sem_s).wait()

        def stage(p, k, first=False):
            q = 1 - p
            wait_idx(q)
            build_gidx(q)
            issue_gather(q, k + 1)
            issue_idx(p, k + 2)
            wait_gather(p)
            compute(p)
            if not first:
                wait_scatter(q)
            build_sidx(q)
            scatter(p)

        issue_idx(0, 0)
        wait_idx(0)
        build_gidx(0)
        build_sidx(0)
        issue_gather(0, 0)
        issue_idx(1, 1)

        stage(0, 0, first=True)
        stage(1, 1)

        def pair2(i, carry):
            k = 2 + i * 2
            stage(0, k)
            stage(1, k + 1)
            return carry
        lax.fori_loop(0, (NCHUNK - 3) // 2, pair2, 0)
        stage(0, NCHUNK - 1)
        wait_gather(1)
        wait_idx(0)
        wait_scatter(0)

        plsc.subcore_barrier()
        for r in range(NWR):
            pltpu.sync_copy(acc.at[pl.ds(s * RPT + r * WR, WR)], zs_v)
            pltpu.sync_copy(zs_v, out_hbm.at[pl.ds(c * NPAD + s * RPT + r * WR, WR)])

    f = pl.kernel(
        body,
        out_type=jax.ShapeDtypeStruct((2 * NPAD, 32), F32),
        mesh=mesh,
        scratch_types=[
            pltpu.VMEM_SHARED((NPAD, 32), F32),
            pltpu.VMEM((2, 2, CH), jnp.int32),
            pltpu.VMEM((2, CH), jnp.int32),
            pltpu.VMEM((2, CH), jnp.int32),
            pltpu.VMEM((2, CH), jnp.int32),
            pltpu.VMEM((2, CH, FEA), jnp.bfloat16),
            pltpu.VMEM((2, CH, FEA), jnp.bfloat16),
            pltpu.VMEM((2, CH, FEA), F32),
            pltpu.VMEM((2, CH, 32), F32),
            pltpu.VMEM((WR, 32), F32),
            pltpu.SemaphoreType.DMA,
            pltpu.SemaphoreType.DMA,
            pltpu.SemaphoreType.DMA,
            pltpu.SemaphoreType.DMA,
            pltpu.SemaphoreType.DMA,
        ],
        compiler_params=pltpu.CompilerParams(use_tc_tiling_on_sc=False, needs_layout_passes=False),
    )
    return f(P, Q, E, idx)


def _sc_count(idx):
    mesh = plsc.VectorSubcoreMesh(core_axis_name="c", subcore_axis_name="s")
    NJ = NCHUNK // 2

    def body(idx_hbm, out_hbm, acc, raw_v, sidx_v, ones_v, zs_v,
             sem_i0, sem_i1, sem_s):
        c = lax.axis_index("c")
        s = lax.axis_index("s")
        zero = jnp.zeros((16,), F32)
        one = jnp.ones((16,), F32)
        sem_i = (sem_i0, sem_i1)

        def zbody(i, carry):
            zs_v[i, pl.ds(0, 16)] = zero
            return carry
        lax.fori_loop(0, WR, zbody, 0)

        def obody(i, carry):
            ones_v[i, pl.ds(0, 16)] = one
            return carry
        lax.fori_loop(0, CH, obody, 0)
        for r in range(NWR):
            pltpu.sync_copy(zs_v, acc.at[pl.ds(s * RPT + r * WR, WR)])
        plsc.subcore_barrier()

        def row_of(j):
            return s * NCHUNK + jnp.minimum(c + 2 * j, NCHUNK - 1)

        def issue_idx(p, j):
            pltpu.async_copy(idx_hbm.at[row_of(j), 0], raw_v.at[p], sem_i[p])

        def wait_idx(p):
            pltpu.make_async_copy(idx_hbm.at[0, 0], raw_v.at[p], sem_i[p]).wait()

        def scatter(p):
            pltpu.async_copy(ones_v, acc.at[sidx_v.at[p]], sem_s, add=True)

        def wait_scatter(p):
            pltpu.make_async_copy(ones_v, acc.at[sidx_v.at[p]], sem_s).wait()

        def stage(p, j, first=False):
            wait_idx(p)
            if not first:
                wait_scatter(p)
            for b in range(CH // 16):
                sl = pl.ds(b * 16, 16)
                sidx_v[p, sl] = raw_v[p, sl]
            scatter(p)
            issue_idx(p, j + 2)

        issue_idx(0, 0)
        issue_idx(1, 1)
        stage(0, 0, first=True)
        stage(1, 1, first=True)

        def pair(i, carry):
            j = 2 + i * 2
            stage(0, j)
            stage(1, j + 1)
            return carry
        lax.fori_loop(0, (NJ - 2) // 2, pair, 0)

        @pl.when(c == 0)
        def _():
            stage(0, NJ)
        wait_idx(0)
        wait_idx(1)
        wait_scatter(0)
        wait_scatter(1)

        plsc.subcore_barrier()
        for r in range(NWR):
            pltpu.sync_copy(acc.at[pl.ds(s * RPT + r * WR, WR)], zs_v)
            pltpu.sync_copy(zs_v, out_hbm.at[pl.ds(c * NPAD + s * RPT + r * WR, WR)])

    f = pl.kernel(
        body,
        out_type=jax.ShapeDtypeStruct((2 * NPAD, 16), F32),
        mesh=mesh,
        scratch_types=[
            pltpu.VMEM_SHARED((NPAD, 16), F32),
            pltpu.VMEM((2, CH), jnp.int32),
            pltpu.VMEM((2, CH), jnp.int32),
            pltpu.VMEM((CH, 16), F32),
            pltpu.VMEM((WR, 16), F32),
            pltpu.SemaphoreType.DMA,
            pltpu.SemaphoreType.DMA,
            pltpu.SemaphoreType.DMA,
        ],
        compiler_params=pltpu.CompilerParams(use_tc_tiling_on_sc=False, needs_layout_passes=False),
    )
    return f(idx)


BLK = 2000
NBLK = N_NODES // BLK
NEBLK = N_EDGES // BLK


def _embed_pq(x, W, b, Wcat0):
    def body(x_ref, w_ref, b_ref, wc_ref, o_ref, p_ref, q_ref):
        x0 = _lrelu(jnp.dot(x_ref[...], w_ref[...],
                            preferred_element_type=F32) + b_ref[...])
        o_ref[...] = x0
        z = jnp.dot(x0, wc_ref[...], preferred_element_type=F32)
        p_ref[0] = z[:, 0:64].astype(jnp.bfloat16)
        p_ref[1] = z[:, 64:128].astype(jnp.bfloat16)
        q_ref[0] = z[:, 128:192].astype(jnp.bfloat16)
        q_ref[1] = z[:, 192:256].astype(jnp.bfloat16)
    return pl.pallas_call(
        body,
        grid=(NBLK,),
        in_specs=[pl.BlockSpec((BLK, NODE_IN), lambda j: (j, 0)),
                  pl.BlockSpec((NODE_IN, FEA), lambda j: (0, 0)),
                  pl.BlockSpec((1, FEA), lambda j: (0, 0)),
                  pl.BlockSpec((FEA, 4 * FEA), lambda j: (0, 0))],
        out_specs=[pl.BlockSpec((BLK, FEA), lambda j: (j, 0)),
                   pl.BlockSpec((2, BLK, FEA), lambda j: (0, j, 0)),
                   pl.BlockSpec((2, BLK, FEA), lambda j: (0, j, 0))],
        out_shape=[jax.ShapeDtypeStruct((N_NODES, FEA), F32),
                   jax.ShapeDtypeStruct((2, N_NODES, FEA), jnp.bfloat16),
                   jax.ShapeDtypeStruct((2, N_NODES, FEA), jnp.bfloat16)],
    )(x, W, b.reshape(1, FEA), Wcat0)


def _edge_E(attr, eW, eb, Wcomb, bcomb):
    def body(a_ref, ew_ref, ebias_ref, wc_ref, bc_ref, o_ref):
        e = _lrelu(jnp.dot(a_ref[...], ew_ref[...],
                           preferred_element_type=F32) + ebias_ref[...])
        for i in range(NL):
            z = jnp.dot(e, wc_ref[i], preferred_element_type=F32) + bc_ref[i]
            ef, es = z[:, :64], z[:, 64:]
            o_ref[i, 0] = jnp.concatenate([ef[:, :32], es[:, :32]], axis=1)
            o_ref[i, 1] = jnp.concatenate([ef[:, 32:], es[:, 32:]], axis=1)
    return pl.pallas_call(
        body,
        grid=(NEBLK,),
        in_specs=[pl.BlockSpec((BLK, 16), lambda j: (j, 0)),
                  pl.BlockSpec((16, FEA), lambda j: (0, 0)),
                  pl.BlockSpec((1, FEA), lambda j: (0, 0)),
                  pl.BlockSpec((NL, FEA, 2 * FEA), lambda j: (0, 0, 0)),
                  pl.BlockSpec((NL, 1, 2 * FEA), lambda j: (0, 0, 0))],
        out_specs=pl.BlockSpec((NL, 2, BLK, FEA), lambda j: (0, 0, j, 0)),
        out_shape=jax.ShapeDtypeStruct((NL, 2, N_EDGES, FEA), F32),
    )(attr, eW, eb.reshape(1, FEA), Wcomb, bcomb)


def _stats(agg0, agg1, cnt0, cnt1):
    def body(a0_ref, a1_ref, c0_ref, c1_ref, o_ref):
        j = pl.program_id(0)
        inv = 1.0 / jnp.maximum(c0_ref[:, 0:1] + c1_ref[:, 0:1], 1.0)
        a = jnp.concatenate([a0_ref[...], a1_ref[...]], axis=1) * inv

        @pl.when(j == 0)
        def _():
            o_ref[...] = jnp.zeros_like(o_ref)
        o_ref[0:1, :] = o_ref[0:1, :] + jnp.sum(a, axis=0, keepdims=True)
        o_ref[1:2, :] = o_ref[1:2, :] + jnp.sum(a * a, axis=0, keepdims=True)
    return pl.pallas_call(
        body,
        grid=(NBLK,),
        in_specs=[pl.BlockSpec((BLK, 32), lambda j: (j, 0)),
                  pl.BlockSpec((BLK, 32), lambda j: (j, 0)),
                  pl.BlockSpec((BLK, 16), lambda j: (j, 0)),
                  pl.BlockSpec((BLK, 16), lambda j: (j, 0))],
        out_specs=pl.BlockSpec((8, FEA), lambda j: (0, 0)),
        out_shape=jax.ShapeDtypeStruct((8, FEA), F32),
    )(agg0, agg1, cnt0, cnt1)


def _apply_pq(agg0, agg1, cnt0, cnt1, x, st, gamma, beta, Wcat_next):
    with_pq = Wcat_next is not None

    def body(a0_ref, a1_ref, c0_ref, c1_ref, x_ref, s_ref, g_ref, b_ref, *rest):
        if with_pq:
            w_ref, o_ref, p_ref, q_ref = rest
        else:
            (o_ref,) = rest
        inv = 1.0 / jnp.maximum(c0_ref[:, 0:1] + c1_ref[:, 0:1], 1.0)
        a = jnp.concatenate([a0_ref[...], a1_ref[...]], axis=1) * inv
        mean = s_ref[0:1, :] / N_NODES
        var = s_ref[1:2, :] / N_NODES - mean * mean
        xn = (a - mean) / jnp.sqrt(var + 1e-5) * g_ref[...] + b_ref[...] + x_ref[...]
        o_ref[...] = xn
        if with_pq:
            z = jnp.dot(xn, w_ref[...], preferred_element_type=F32)
            p_ref[0] = z[:, 0:64].astype(jnp.bfloat16)
            p_ref[1] = z[:, 64:128].astype(jnp.bfloat16)
            q_ref[0] = z[:, 128:192].astype(jnp.bfloat16)
            q_ref[1] = z[:, 192:256].astype(jnp.bfloat16)

    in_specs = [pl.BlockSpec((BLK, 32), lambda j: (j, 0)),
                pl.BlockSpec((BLK, 32), lambda j: (j, 0)),
                pl.BlockSpec((BLK, 16), lambda j: (j, 0)),
                pl.BlockSpec((BLK, 16), lambda j: (j, 0)),
                pl.BlockSpec((BLK, FEA), lambda j: (j, 0)),
                pl.BlockSpec((8, FEA), lambda j: (0, 0)),
                pl.BlockSpec((1, FEA), lambda j: (0, 0)),
                pl.BlockSpec((1, FEA), lambda j: (0, 0))]
    out_specs = [pl.BlockSpec((BLK, FEA), lambda j: (j, 0))]
    out_shape = [jax.ShapeDtypeStruct((N_NODES, FEA), F32)]
    args = [agg0, agg1, cnt0, cnt1, x, st, gamma.reshape(1, FEA), beta.reshape(1, FEA)]
    if with_pq:
        in_specs.append(pl.BlockSpec((FEA, 4 * FEA), lambda j: (0, 0)))
        out_specs += [pl.BlockSpec((2, BLK, FEA), lambda j: (0, j, 0)),
                      pl.BlockSpec((2, BLK, FEA), lambda j: (0, j, 0))]
        out_shape += [jax.ShapeDtypeStruct((2, N_NODES, FEA), jnp.bfloat16),
                      jax.ShapeDtypeStruct((2, N_NODES, FEA), jnp.bfloat16)]
        args.append(Wcat_next)
    res = pl.pallas_call(
        body,
        grid=(NBLK,),
        in_specs=in_specs,
        out_specs=out_specs,
        out_shape=out_shape,
    )(*args)
    return res if with_pq else (res[0], None, None)


def _gate_max(x, batch, gW1, gb1, gW2, gb2):
    def body(x_ref, b_ref, w1_ref, b1_ref, w2_ref, b2_ref, gate_ref, m_ref):
        j = pl.program_id(0)
        g1 = jnp.dot(x_ref[...], w1_ref[...], preferred_element_type=F32) + b1_ref[...]
        gate = jnp.dot(g1, w2_ref[...], preferred_element_type=F32) + b2_ref[...]
        gate_ref[...] = gate
        gid = lax.broadcasted_iota(jnp.int32, (1, NG), 1)
        M = b_ref[...] == gid
        masked = jnp.where(M, gate, -3e38)
        bm = jnp.max(masked, axis=0, keepdims=True)

        @pl.when(j == 0)
        def _():
            m_ref[...] = jnp.full_like(m_ref, -3e38)
        m_ref[...] = jnp.maximum(m_ref[...], bm)
    return pl.pallas_call(
        body,
        grid=(NBLK,),
        in_specs=[pl.BlockSpec((BLK, FEA), lambda j: (j, 0)),
                  pl.BlockSpec((BLK, 1), lambda j: (j, 0)),
                  pl.BlockSpec((FEA, FEA), lambda j: (0, 0)),
                  pl.BlockSpec((1, FEA), lambda j: (0, 0)),
                  pl.BlockSpec((FEA, 1), lambda j: (0, 0)),
                  pl.BlockSpec((1, 1), lambda j: (0, 0))],
        out_specs=[pl.BlockSpec((BLK, 1), lambda j: (j, 0)),
                   pl.BlockSpec((1, NG), lambda j: (0, 0))],
        out_shape=[jax.ShapeDtypeStruct((N_NODES, 1), F32),
                   jax.ShapeDtypeStruct((1, NG), F32)],
    )(x, batch, gW1, gb1.reshape(1, FEA), gW2, gb2.reshape(1, 1))


def _pool_acc(x, batch, gate, m, nW1, nb1, nW2, nb2):
    def body(x_ref, b_ref, gate_ref, m_ref, w1_ref, b1_ref, w2_ref, b2_ref,
             s_ref, n_ref):
        j = pl.program_id(0)
        h1 = jnp.dot(x_ref[...], w1_ref[...], preferred_element_type=F32) + b1_ref[...]
        h = jnp.dot(h1, w2_ref[...], preferred_element_type=F32) + b2_ref[...]
        gid = lax.broadcasted_iota(jnp.int32, (1, NG), 1)
        M = (b_ref[...] == gid).astype(F32)
        mb = jnp.sum(M * m_ref[...], axis=1, keepdims=True)
        g = jnp.exp(gate_ref[...] - mb)
        Mg = M * g
        s_blk = jnp.sum(Mg, axis=0, keepdims=True)
        n_blk = lax.dot_general(h, Mg, (((0,), (0,)), ((), ())),
                                preferred_element_type=F32)

        @pl.when(j == 0)
        def _():
            s_ref[...] = jnp.zeros_like(s_ref)
            n_ref[...] = jnp.zeros_like(n_ref)
        s_ref[...] = s_ref[...] + s_blk
        n_ref[...] = n_ref[...] + n_blk
    return pl.pallas_call(
        body,
        grid=(NBLK,),
        in_specs=[pl.BlockSpec((BLK, FEA), lambda j: (j, 0)),
                  pl.BlockSpec((BLK, 1), lambda j: (j, 0)),
                  pl.BlockSpec((BLK, 1), lambda j: (j, 0)),
                  pl.BlockSpec((1, NG), lambda j: (0, 0)),
                  pl.BlockSpec((FEA, FEA), lambda j: (0, 0)),
                  pl.BlockSpec((1, FEA), lambda j: (0, 0)),
                  pl.BlockSpec((FEA, FEA), lambda j: (0, 0)),
                  pl.BlockSpec((1, FEA), lambda j: (0, 0))],
        out_specs=[pl.BlockSpec((1, NG), lambda j: (0, 0)),
                   pl.BlockSpec((FEA, NG), lambda j: (0, 0))],
        out_shape=[jax.ShapeDtypeStruct((1, NG), F32),
                   jax.ShapeDtypeStruct((FEA, NG), F32)],
    )(x, batch, gate, m, nW1, nb1.reshape(1, FEA), nW2, nb2.reshape(1, FEA))


def _heads(Ne, se, Ns, ss, h_W, h_b, out_W, out_b, hexp_W, hexp_b):
    def body(ne_ref, se_ref, ns_ref, ss_ref, hw_ref, hb_ref, ow_ref, ob_ref,
             xw_ref, xb_ref, eo_ref, so_ref):
        pe = ne_ref[...] / jnp.maximum(se_ref[...], 1e-30)
        ps = ns_ref[...] / jnp.maximum(ss_ref[...], 1e-30)
        so1 = _lrelu(lax.dot_general(hw_ref[...], ps, (((0,), (0,)), ((), ())),
                                     preferred_element_type=F32) + hb_ref[...])
        so_ref[...] = lax.dot_general(ow_ref[...], so1, (((0,), (0,)), ((), ())),
                                      preferred_element_type=F32) + ob_ref[...]
        eo_ref[...] = lax.dot_general(xw_ref[...], pe, (((0,), (0,)), ((), ())),
                                      preferred_element_type=F32) + xb_ref[...]
    return pl.pallas_call(
        body,
        out_shape=[jax.ShapeDtypeStruct((1, NG), F32),
                   jax.ShapeDtypeStruct((1, NG), F32)],
    )(Ne, se, Ns, ss, h_W, h_b.reshape(FEA, 1), out_W, out_b.reshape(1, 1),
      hexp_W, hexp_b.reshape(1, 1))


def kernel(exp_x, exp_edge_index, exp_edge_attr, exp_batch, sim_x,
           sim_edge_index, sim_edge_attr, sim_batch, node_W, node_b, edge_W,
           edge_b, linf_W, linf_b, lins_W, lins_b, bn_g, bn_b, gate_W1,
           gate_b1, gate_W2, gate_b2, nn_W1, nn_b1, nn_W2, nn_b2, h_W, h_b,
           out_W, out_b, hexp_W, hexp_b):
    dst_e = exp_edge_index[1]
    src_e = exp_edge_index[0]
    dst_s = sim_edge_index[1]
    src_s = sim_edge_index[0]
    batch_e = exp_batch.reshape(N_NODES, 1)
    batch_s = sim_batch.reshape(N_NODES, 1)

    def _ilv(base):
        out = []
        for j in range(16):
            out += [base + j, base + 16 + j]
        return out
    perm_pq = []
    for sec in (0, 128):
        for c2 in (0, 1):
            perm_pq += _ilv(sec + c2 * 32) + _ilv(sec + 64 + c2 * 32)
    perm_pq = jnp.array(perm_pq, dtype=jnp.int32)
    Wcat = [jnp.concatenate([linf_W[i, 0:64], lins_W[i, 0:64],
                             linf_W[i, 64:128], lins_W[i, 64:128]],
                            axis=1)[:, perm_pq]
            for i in range(NL)]
    Wcomb = jnp.stack([jnp.concatenate([linf_W[i, 128:192], lins_W[i, 128:192]],
                                       axis=1) for i in range(NL)])
    bcomb = jnp.stack([jnp.concatenate([linf_b[i], lins_b[i]]).reshape(1, 2 * FEA)
                       for i in range(NL)])

    idx_e = jnp.stack([dst_e.reshape(-1, CH), src_e.reshape(-1, CH)], axis=1)
    idx_s = jnp.stack([dst_s.reshape(-1, CH), src_s.reshape(-1, CH)], axis=1)
    xe, Pe, Qe = _embed_pq(exp_x, node_W, node_b, Wcat[0])
    xs, Ps, Qs = _embed_pq(sim_x, node_W, node_b, Wcat[0])
    Ee = _edge_E(exp_edge_attr, edge_W, edge_b, Wcomb, bcomb)
    Es = _edge_E(sim_edge_attr, edge_W, edge_b, Wcomb, bcomb)
    Ee = Ee.reshape(NL * 2 * N_EDGES, FEA)
    Es = Es.reshape(NL * 2 * N_EDGES, FEA)
    cnt_e = _sc_count(idx_e)
    cnt_s = _sc_count(idx_s)

    state = dict(
        e=[xe, Pe, Qe, Ee, idx_e, cnt_e],
        s=[xs, Ps, Qs, Es, idx_s, cnt_s],
    )
    for i in range(NL):
        for tag in ("e", "s"):
            x, P, Q, E, idx, cnt = state[tag]
            agg = _sc_edge_layer(i, P.reshape(2 * N_NODES, FEA),
                                 Q.reshape(2 * N_NODES, FEA), E, idx)
            agg0, agg1 = agg[:NPAD], agg[NPAD:]
            st = _stats(agg0, agg1, cnt[:NPAD], cnt[NPAD:])
            Wn = Wcat[i + 1] if i + 1 < NL else None
            xn, Pn, Qn = _apply_pq(agg0, agg1, cnt[:NPAD], cnt[NPAD:], x, st,
                                   bn_g[i], bn_b[i], Wn)
            state[tag] = [xn, Pn, Qn, E, idx, cnt]
    xe = state["e"][0]
    xs = state["s"][0]

    gate_e, m_e = _gate_max(xe, batch_e, gate_W1, gate_b1, gate_W2, gate_b2)
    gate_s, m_s = _gate_max(xs, batch_s, gate_W1, gate_b1, gate_W2, gate_b2)
    s_e, N_e = _pool_acc(xe, batch_e, gate_e, m_e, nn_W1, nn_b1, nn_W2, nn_b2)
    s_s, N_s = _pool_acc(xs, batch_s, gate_s, m_s, nn_W1, nn_b1, nn_W2, nn_b2)
    eo, so = _heads(N_e, s_e, N_s, s_s, h_W, h_b, out_W, out_b, hexp_W, hexp_b)
    return (eo.reshape(NG), so.reshape(NG))
